# Initial kernel scaffold; baseline (speedup 1.0000x reference)
import jax
import jax.numpy as jnp
from jax import lax
from jax.experimental import pallas as pl
from jax.experimental.pallas import tpu as pltpu, tpu_sc as plsc

VOCAB = 1000000
D = 64
NC, NS = 2, 16
NW = NC * NS
CHUNK = 128


def _build(batch_rows: int):
    chunks = batch_rows // (NW * CHUNK)
    b_per_w = chunks * CHUNK
    mesh = plsc.VectorSubcoreMesh(
        core_axis_name="c", subcore_axis_name="s", num_cores=NC, num_subcores=NS
    )

    def body(x_hbm, table_hbm, out_hbm, idx_v, rows_v, sem):
        wid = lax.axis_index("c") * NS + lax.axis_index("s")
        pltpu.sync_copy(x_hbm.at[wid], idx_v)
        base = wid * b_per_w

        def step(j, carry):
            pltpu.async_copy(table_hbm.at[idx_v.at[j]], rows_v, sem).wait()
            pltpu.sync_copy(rows_v, out_hbm.at[pl.ds(base + j * CHUNK, CHUNK)])
            return carry

        lax.fori_loop(0, chunks, step, 0)

    return pl.kernel(
        body,
        out_type=jax.ShapeDtypeStruct((batch_rows, D), jnp.float32),
        mesh=mesh,
        scratch_types=[
            pltpu.VMEM((chunks, CHUNK), jnp.int32),
            pltpu.VMEM((CHUNK, D), jnp.float32),
            pltpu.SemaphoreType.DMA,
        ],
    )


def kernel(x, table):
    b, h = x.shape
    rows = b * h
    idx = x.reshape(NW, rows // (NW * CHUNK), CHUNK)
    out = _build(rows)(idx, table)
    return out.reshape(b, h, D)


# speedup vs baseline: 1.6838x; 1.6838x over previous
import jax
import jax.numpy as jnp
from jax import lax
from jax.experimental import pallas as pl
from jax.experimental.pallas import tpu as pltpu, tpu_sc as plsc

VOCAB = 1000000
D = 64
NC, NS = 2, 16
NW = NC * NS
CHUNK = 128


def _build(batch_rows: int):
    chunks = batch_rows // (NW * CHUNK)
    b_per_w = chunks * CHUNK
    mesh = plsc.VectorSubcoreMesh(
        core_axis_name="c", subcore_axis_name="s", num_cores=NC, num_subcores=NS
    )

    def body(x_hbm, table_hbm, out_hbm, idx_v, rows_v, sem):
        wid = lax.axis_index("c") * NS + lax.axis_index("s")
        pltpu.sync_copy(x_hbm.at[wid], idx_v)
        base = wid * b_per_w

        def step(j, carry):
            pltpu.async_copy(table_hbm.at[idx_v.at[j]], rows_v, sem).wait()
            pltpu.sync_copy(rows_v, out_hbm.at[pl.ds(base + j * CHUNK, CHUNK)])
            return carry

        lax.fori_loop(0, chunks, step, 0)

    return pl.kernel(
        body,
        out_type=jax.ShapeDtypeStruct((batch_rows, D), jnp.float32),
        mesh=mesh,
        scratch_types=[
            pltpu.VMEM((chunks, CHUNK), jnp.int32),
            pltpu.VMEM((CHUNK, D), jnp.float32),
            pltpu.SemaphoreType.DMA,
        ],
        compiler_params=pltpu.CompilerParams(use_tc_tiling_on_sc=False),
    )


def kernel(x, table):
    b, h = x.shape
    rows = b * h
    idx = x.reshape(NW, rows // (NW * CHUNK), CHUNK)
    out = _build(rows)(idx, table)
    return out.reshape(b, h, D)


# speedup vs baseline: 1.8889x; 1.1218x over previous
import jax
import jax.numpy as jnp
from jax import lax
from jax.experimental import pallas as pl
from jax.experimental.pallas import tpu as pltpu, tpu_sc as plsc

VOCAB = 1000000
D = 64
NC, NS = 2, 16
NW = NC * NS
CHUNK = 128


K = 4
SUPER = K * CHUNK


def _build(batch_rows: int):
    chunks = batch_rows // (NW * CHUNK)
    b_per_w = chunks * CHUNK
    nsuper = chunks // K
    mesh = plsc.VectorSubcoreMesh(
        core_axis_name="c", subcore_axis_name="s", num_cores=NC, num_subcores=NS
    )

    def body(x_hbm, table_hbm, out_hbm, idx_v, buf_a, buf_b, gsem, wsem):
        wid = lax.axis_index("c") * NS + lax.axis_index("s")
        pltpu.sync_copy(x_hbm.at[wid], idx_v)
        base = wid * b_per_w
        bufs = (buf_a, buf_b)

        def fire(g, buf):
            for k in range(K):
                j = g * K + k
                pltpu.async_copy(
                    table_hbm.at[idx_v.at[j]],
                    buf.at[pl.ds(k * CHUNK, CHUNK)],
                    gsem,
                )

        def drain_gathers(buf):
            pltpu.make_async_copy(table_hbm.at[idx_v.at[0]], buf, gsem).wait()

        def write(g, buf):
            pltpu.async_copy(buf, out_hbm.at[pl.ds(base + g * SUPER, SUPER)], wsem)

        def wait_write(g, buf):
            pltpu.make_async_copy(buf, out_hbm.at[pl.ds(base + g * SUPER, SUPER)], wsem).wait()

        fire(0, buf_a)

        def step(i, carry):
            for b in range(2):
                g = i * 2 + b
                nxt = bufs[1 - b]

                @pl.when(g >= 1)
                def _():
                    wait_write(g - 1, nxt)

                @pl.when(g + 1 < nsuper)
                def _():
                    fire(g + 1, nxt)

                drain_gathers(bufs[b])
                write(g, bufs[b])
            return carry

        lax.fori_loop(0, nsuper // 2, step, 0)
        wait_write(nsuper - 1, buf_b)

    return pl.kernel(
        body,
        out_type=jax.ShapeDtypeStruct((batch_rows, D), jnp.float32),
        mesh=mesh,
        scratch_types=[
            pltpu.VMEM((chunks, CHUNK), jnp.int32),
            pltpu.VMEM((SUPER, D), jnp.float32),
            pltpu.VMEM((SUPER, D), jnp.float32),
            pltpu.SemaphoreType.DMA,
            pltpu.SemaphoreType.DMA,
        ],
        compiler_params=pltpu.CompilerParams(use_tc_tiling_on_sc=False),
    )


def kernel(x, table):
    b, h = x.shape
    rows = b * h
    idx = x.reshape(NW, rows // (NW * CHUNK), CHUNK)
    out = _build(rows)(idx, table)
    return out.reshape(b, h, D)
